# baseline (speedup 1.0000x reference)
import functools

import numpy as np

import jax
import jax.numpy as jnp
from jax import lax
from jax.experimental import pallas as pl
from jax.experimental.pallas import tpu as pltpu
from jax.experimental.pallas import tpu_sc as plsc

_N = 160000
_G = 512
_NT = 128
_NK = 129
_NW = 32
_CHUNK = _N // _NW
_SBUF = 5008
_ITER = _SBUF // 16
_PBUF = 10000
_PITER = _PBUF // 16
_NEG = float("-inf")


_PERM_BIG = np.concatenate([np.arange(b, b + _NT) for b in
                            (0, 128, 259, 387, 518, 646)]).astype(np.int32)
_B128 = np.array([256, 515, 774], dtype=np.int32)
_SMALL = np.array([257, 258, 516, 517, 775, 776], dtype=np.int32)


def _take(x, idx):
  return x.at[idx].get(mode="promise_in_bounds")


def _sc_pool_body(bat_h, typ_h, tok_h, xsv_h,
                  out_t, out_k, out_sm,
                  b_v, t_v, k_v, sv_v, hist_t, hist_k, smacc):
  cid = lax.axis_index("c")
  sid = lax.axis_index("s")
  wid = sid * 2 + cid
  base = wid * _CHUNK
  pltpu.sync_copy(bat_h.at[pl.ds(base, _CHUNK)], b_v.at[pl.ds(0, _CHUNK)])
  pltpu.sync_copy(typ_h.at[pl.ds(base, _CHUNK)], t_v.at[pl.ds(0, _CHUNK)])
  pltpu.sync_copy(tok_h.at[pl.ds(base, _CHUNK)], k_v.at[pl.ds(0, _CHUNK)])
  pltpu.sync_copy(xsv_h.at[pl.ds(2 * base, 2 * _CHUNK)],
                  sv_v.at[pl.ds(0, 2 * _CHUNK)])

  lanes = lax.iota(jnp.int32, 16)
  izero16 = jnp.zeros((16,), jnp.int32)
  zero16 = jnp.zeros((16,), jnp.float32)
  ninf16 = jnp.full((16,), _NEG, jnp.float32)

  def _zero_hists(j, c):
    for q in range(8):
      hist_t[j, pl.ds(q * 16, 16)] = izero16
      hist_k[j, pl.ds(q * 16, 16)] = izero16
    return c

  lax.fori_loop(0, _G // 2, _zero_hists, 0)
  for r in range(16):
    val = ninf16 if r in (4, 5) else zero16
    for q in range(_G // 16):
      smacc[r, pl.ds(q * 16, 16)] = val

  def _hist_loop(i, c):
    off = i * 16
    valid = (off + lanes) < _CHUNK
    b = jnp.where(valid, b_v[pl.ds(off, 16)], -1)
    t = t_v[pl.ds(off, 16)]
    k = jnp.clip(k_v[pl.ds(off, 16)], 0, _NK - 1)
    odd = (b & 1) == 1
    row = lax.shift_right_logical(b, 1)
    cnt_t, last_t = plsc.scan_count(b * _NT + t, mask=valid)
    plsc.addupdate_scatter(hist_t, [row, t], cnt_t,
                           mask=last_t & jnp.logical_not(odd))
    plsc.addupdate_scatter(hist_t, [row, t], lax.shift_left(cnt_t, 16),
                           mask=last_t & odd)
    cnt_k, last_k = plsc.scan_count(b * 256 + k, mask=valid)
    inb = last_k & (k < _NT)
    plsc.addupdate_scatter(hist_k, [row, k], cnt_k,
                           mask=inb & jnp.logical_not(odd))
    plsc.addupdate_scatter(hist_k, [row, k], lax.shift_left(cnt_k, 16),
                           mask=inb & odd)
    return c

  lax.fori_loop(0, _ITER, _hist_loop, 0)
  pltpu.sync_copy(hist_t, out_t.at[wid])
  pltpu.sync_copy(hist_k, out_k.at[wid])

  def _small_loop(i, c):
    off = i * 16
    bp = _take(b_v[pl.ds(off // 2, 16)], lax.shift_right_logical(lanes, 1))
    par = lanes & 1
    v = sv_v[pl.ds(off, 16)]
    s = v
    q = v * v
    c1 = jnp.full((16,), 0.0, jnp.float32) + 1.0
    m = v
    for d in (2, 4, 8):
      sidx = jnp.maximum(lanes - d, 0)
      bd = _take(bp, sidx)
      same = (bd == bp) & (lanes >= d)
      c1 = c1 + jnp.where(same, _take(c1, sidx), 0.0)
      s = s + jnp.where(same, _take(s, sidx), 0.0)
      q = q + jnp.where(same, _take(q, sidx), 0.0)
      m = jnp.maximum(m, jnp.where(same, _take(m, sidx), _NEG))
    nb = _take(bp, jnp.minimum(lanes + 2, 15))
    lastseg = (bp != nb) | (lanes >= 14)
    plsc.addupdate_scatter(smacc, [par, bp], s, mask=lastseg)
    plsc.addupdate_scatter(smacc, [par + 2, bp], q, mask=lastseg)
    plsc.addupdate_scatter(smacc, [par * 0 + 6, bp], c1,
                           mask=lastseg & (par == 0))
    cur = plsc.load_gather(smacc, [par + 4, bp], mask=lastseg)
    plsc.store_scatter(smacc, [par + 4, bp], jnp.maximum(cur, m), mask=lastseg)
    return c

  lax.fori_loop(0, _PITER, _small_loop, 0)
  pltpu.sync_copy(smacc, out_sm.at[wid])


_sc_pool = functools.partial(
    pl.kernel,
    out_type=[
        jax.ShapeDtypeStruct((_NW, _G // 2, _NT), jnp.int32),
        jax.ShapeDtypeStruct((_NW, _G // 2, _NT), jnp.int32),
        jax.ShapeDtypeStruct((_NW, 16, _G), jnp.float32),
    ],
    mesh=plsc.VectorSubcoreMesh(core_axis_name="c", subcore_axis_name="s"),
    compiler_params=pltpu.CompilerParams(needs_layout_passes=False),
    scratch_types=[
        pltpu.VMEM((_SBUF,), jnp.int32),
        pltpu.VMEM((_SBUF,), jnp.int32),
        pltpu.VMEM((_SBUF,), jnp.int32),
        pltpu.VMEM((_PBUF,), jnp.float32),
        pltpu.VMEM((_G // 2, _NT), jnp.int32),
        pltpu.VMEM((_G // 2, _NT), jnp.int32),
        pltpu.VMEM((16, _G), jnp.float32),
    ],
)(_sc_pool_body)


def _leaky(v):
  return jnp.where(v > 0, v, 0.01 * v)


def _tc_mlp_body(ht, hk, sm, w1b, w1c, w1s, b1, w2, b2, w3, b3, out,
                 alo_t, ahi_t, alo_k, ahi_k, acc_s, acc_m, acc_c):
  i = pl.program_id(0)

  @pl.when(i == 0)
  def _():
    alo_t[...] = jnp.zeros_like(alo_t)
    ahi_t[...] = jnp.zeros_like(ahi_t)
    alo_k[...] = jnp.zeros_like(alo_k)
    ahi_k[...] = jnp.zeros_like(ahi_k)
    acc_s[...] = jnp.zeros_like(acc_s)
    acc_m[...] = jnp.full_like(acc_m, _NEG)
    acc_c[...] = jnp.zeros_like(acc_c)

  lo_t = alo_t[...]
  hi_t = ahi_t[...]
  lo_k = alo_k[...]
  hi_k = ahi_k[...]
  s_s = acc_s[...]
  s_m = acc_m[...]
  s_c = acc_c[...]
  for j in range(8):
    wt = ht[j]
    wk = hk[j]
    lo_t += jnp.bitwise_and(wt, 0xFFFF).astype(jnp.float32)
    hi_t += lax.shift_right_logical(wt, 16).astype(jnp.float32)
    lo_k += jnp.bitwise_and(wk, 0xFFFF).astype(jnp.float32)
    hi_k += lax.shift_right_logical(wk, 16).astype(jnp.float32)
    s_s += sm[j, 0:4, :]
    s_m = jnp.maximum(s_m, sm[j, 4:6, :])
    s_c += sm[j, 6:7, :]
  alo_t[...] = lo_t
  ahi_t[...] = hi_t
  alo_k[...] = lo_k
  ahi_k[...] = hi_k
  acc_s[...] = s_s
  acc_m[...] = s_m
  acc_c[...] = s_c

  @pl.when(i == 3)
  def _():
    at = jnp.stack([alo_t[...], ahi_t[...]], axis=1).reshape(_G, _NT)
    ak = jnp.stack([alo_k[...], ahi_k[...]], axis=1).reshape(_G, _NT)
    cnt = jnp.sum(at, axis=1, keepdims=True)
    cntc = jnp.maximum(cnt, 1.0)
    empty = cnt <= 0.0
    mt = at / cntc
    mk = ak / cntc
    xt = jnp.where(empty, _NEG, (at > 0).astype(jnp.float32))
    xk = jnp.where(empty, _NEG, (ak > 0).astype(jnp.float32))
    st = jnp.sqrt(jnp.clip(mt - mt * mt, 0.0, None) + 1e-8)
    sk = jnp.sqrt(jnp.clip(mk - mk * mk, 0.0, None) + 1e-8)
    k128 = cnt - jnp.sum(ak, axis=1, keepdims=True)
    mk1 = k128 / cntc
    xk1 = jnp.where(empty, _NEG, (k128 > 0).astype(jnp.float32))
    sk1 = jnp.sqrt(jnp.clip(mk1 - mk1 * mk1, 0.0, None) + 1e-8)

    sums = acc_s[...]
    maxs = acc_m[...]
    cnt_r = jnp.maximum(acc_c[...], 1.0)
    ms = sums[0:2, :] / cnt_r
    qs = sums[2:4, :] / cnt_r
    ss = jnp.sqrt(jnp.clip(qs - ms * ms, 0.0, None) + 1e-8)
    small_f = jnp.concatenate([ms, maxs, ss], axis=0)

    hbig = jnp.concatenate([mt, mk, xt, xk, st, sk], axis=1)
    hb = jnp.concatenate([mk1, xk1, sk1], axis=1)
    z1 = jnp.dot(hbig, w1b[...], preferred_element_type=jnp.float32)
    z1 += jnp.dot(hb, w1c[...], preferred_element_type=jnp.float32)
    z1 += lax.dot_general(small_f, w1s[...], (((0,), (0,)), ((), ())),
                          preferred_element_type=jnp.float32)
    h1 = _leaky(z1 + b1[...])
    h2 = _leaky(jnp.dot(h1, w2[...], preferred_element_type=jnp.float32) + b2[...])
    out[...] = jnp.sum(h2 * w3[...], axis=1, keepdims=True) + b3[...]


def _tc_mlp(ht, hk, sm, w1b, w1c, w1s, b1, w2, b2, w3row, b3):
  return pl.pallas_call(
      _tc_mlp_body,
      grid=(4,),
      in_specs=[
          pl.BlockSpec((8, _G // 2, _NT), lambda i: (i, 0, 0)),
          pl.BlockSpec((8, _G // 2, _NT), lambda i: (i, 0, 0)),
          pl.BlockSpec((8, 16, _G), lambda i: (i, 0, 0)),
          pl.BlockSpec((768, 256), lambda i: (0, 0)),
          pl.BlockSpec((3, 256), lambda i: (0, 0)),
          pl.BlockSpec((6, 256), lambda i: (0, 0)),
          pl.BlockSpec((1, 256), lambda i: (0, 0)),
          pl.BlockSpec((256, 256), lambda i: (0, 0)),
          pl.BlockSpec((1, 256), lambda i: (0, 0)),
          pl.BlockSpec((1, 256), lambda i: (0, 0)),
          pl.BlockSpec((1, 1), lambda i: (0, 0)),
      ],
      out_specs=pl.BlockSpec((_G, 1), lambda i: (0, 0)),
      out_shape=jax.ShapeDtypeStruct((_G, 1), jnp.float32),
      scratch_shapes=[
          pltpu.VMEM((_G // 2, _NT), jnp.float32),
          pltpu.VMEM((_G // 2, _NT), jnp.float32),
          pltpu.VMEM((_G // 2, _NT), jnp.float32),
          pltpu.VMEM((_G // 2, _NT), jnp.float32),
          pltpu.VMEM((4, _G), jnp.float32),
          pltpu.VMEM((2, _G), jnp.float32),
          pltpu.VMEM((1, _G), jnp.float32),
      ],
      compiler_params=pltpu.CompilerParams(
          dimension_semantics=("arbitrary",)),
  )(ht, hk, sm, w1b, w1c, w1s, b1, w2, b2, w3row, b3)


def kernel(x_type, x_tok, x_small, batch, W1, b1, W2, b2, W3, b3):
  bat = batch.astype(jnp.int32)
  typ = x_type.astype(jnp.int32)
  tok = x_tok.astype(jnp.int32)
  xsv = x_small.astype(jnp.float32).reshape(-1)
  ht, hk, sm = _sc_pool(bat, typ, tok, xsv)
  out = _tc_mlp(
      ht,
      hk,
      sm,
      W1[_PERM_BIG],
      W1[_B128],
      W1[_SMALL],
      b1.reshape(1, 256),
      W2,
      b2.reshape(1, 256),
      W3.reshape(1, 256),
      b3.reshape(1, 1),
  )
  return out.reshape(-1)


# speedup vs baseline: 2.2993x; 2.2993x over previous
import functools

import numpy as np

import jax
import jax.numpy as jnp
from jax import lax
from jax.experimental import pallas as pl
from jax.experimental.pallas import tpu as pltpu
from jax.experimental.pallas import tpu_sc as plsc

_N = 160000
_G = 512
_NT = 128
_NK = 129
_NW = 32
_CHUNK = _N // _NW
_SBUF = 5008
_ITER = _SBUF // 16
_NEG = float("-inf")


_PERM_BIG = np.concatenate([np.arange(b, b + _NT) for b in
                            (0, 128, 259, 387, 518, 646)]).astype(np.int32)
_B128 = np.array([256, 515, 774], dtype=np.int32)
_SMALL = np.array([257, 258, 516, 517, 775, 776], dtype=np.int32)


def _take(x, idx):
  return x.at[idx].get(mode="promise_in_bounds")


def _sc_pool_body(bat_h, typ_h, tok_h, xs0_h, xs1_h,
                  out_t, out_k, out_sm,
                  b_v, t_v, k_v, s0_v, s1_v, hist_t, hist_k, smacc):
  cid = lax.axis_index("c")
  sid = lax.axis_index("s")
  wid = sid * 2 + cid
  base = wid * _CHUNK
  pltpu.sync_copy(bat_h.at[pl.ds(base, _CHUNK)], b_v.at[pl.ds(0, _CHUNK)])
  pltpu.sync_copy(typ_h.at[pl.ds(base, _CHUNK)], t_v.at[pl.ds(0, _CHUNK)])
  pltpu.sync_copy(tok_h.at[pl.ds(base, _CHUNK)], k_v.at[pl.ds(0, _CHUNK)])
  pltpu.sync_copy(xs0_h.at[pl.ds(base, _CHUNK)], s0_v.at[pl.ds(0, _CHUNK)])
  pltpu.sync_copy(xs1_h.at[pl.ds(base, _CHUNK)], s1_v.at[pl.ds(0, _CHUNK)])

  lanes = lax.iota(jnp.int32, 16)
  izero16 = jnp.zeros((16,), jnp.int32)
  zero16 = jnp.zeros((16,), jnp.float32)
  ninf16 = jnp.full((16,), _NEG, jnp.float32)

  def _zero_hists(j, c):
    for q in range(8):
      hist_t[j, pl.ds(q * 16, 16)] = izero16
      hist_k[j, pl.ds(q * 16, 16)] = izero16
    return c

  lax.fori_loop(0, _G // 2, _zero_hists, 0)
  for r in range(16):
    val = ninf16 if r in (4, 5) else zero16
    for q in range(_G // 16):
      smacc[r, pl.ds(q * 16, 16)] = val

  def _hist_loop(i, c):
    off = i * 16
    valid = (off + lanes) < _CHUNK
    b = jnp.where(valid, b_v[pl.ds(off, 16)], -1)
    t = t_v[pl.ds(off, 16)]
    k = jnp.clip(k_v[pl.ds(off, 16)], 0, _NK - 1)
    odd = (b & 1) == 1
    row = lax.shift_right_logical(b, 1)
    cnt_t, last_t = plsc.scan_count(b * _NT + t, mask=valid)
    plsc.addupdate_scatter(hist_t, [row, t], cnt_t,
                           mask=last_t & jnp.logical_not(odd))
    plsc.addupdate_scatter(hist_t, [row, t], lax.shift_left(cnt_t, 16),
                           mask=last_t & odd)
    cnt_k, last_k = plsc.scan_count(b * 256 + k, mask=valid)
    inb = last_k & (k < _NT)
    plsc.addupdate_scatter(hist_k, [row, k], cnt_k,
                           mask=inb & jnp.logical_not(odd))
    plsc.addupdate_scatter(hist_k, [row, k], lax.shift_left(cnt_k, 16),
                           mask=inb & odd)
    return c

  lax.fori_loop(0, _ITER, _hist_loop, 0)
  pltpu.sync_copy(hist_t, out_t.at[wid])
  pltpu.sync_copy(hist_k, out_k.at[wid])

  def _small_loop(i, c):
    off = i * 16
    valid = (off + lanes) < _CHUNK
    b = jnp.where(valid, b_v[pl.ds(off, 16)], -1)
    v0 = jnp.where(valid, s0_v[pl.ds(off, 16)], 0.0)
    v1 = jnp.where(valid, s1_v[pl.ds(off, 16)], 0.0)
    s0 = v0
    s1 = v1
    q0 = v0 * v0
    q1 = v1 * v1
    c1 = jnp.where(valid, 1.0, 0.0)
    m0 = jnp.where(valid, v0, _NEG)
    m1 = jnp.where(valid, v1, _NEG)
    for d in (1, 2, 4, 8):
      sidx = jnp.maximum(lanes - d, 0)
      bd = _take(b, sidx)
      same = (bd == b) & (lanes >= d)
      c1 = c1 + jnp.where(same, _take(c1, sidx), 0.0)
      s0 = s0 + jnp.where(same, _take(s0, sidx), 0.0)
      s1 = s1 + jnp.where(same, _take(s1, sidx), 0.0)
      q0 = q0 + jnp.where(same, _take(q0, sidx), 0.0)
      q1 = q1 + jnp.where(same, _take(q1, sidx), 0.0)
      m0 = jnp.maximum(m0, jnp.where(same, _take(m0, sidx), _NEG))
      m1 = jnp.maximum(m1, jnp.where(same, _take(m1, sidx), _NEG))
    nb = _take(b, jnp.minimum(lanes + 1, 15))
    lastseg = ((b != nb) | (lanes == 15)) & valid
    r0 = lanes * 0
    plsc.addupdate_scatter(smacc, [r0, b], s0, mask=lastseg)
    plsc.addupdate_scatter(smacc, [r0 + 1, b], s1, mask=lastseg)
    plsc.addupdate_scatter(smacc, [r0 + 2, b], q0, mask=lastseg)
    plsc.addupdate_scatter(smacc, [r0 + 3, b], q1, mask=lastseg)
    plsc.addupdate_scatter(smacc, [r0 + 6, b], c1, mask=lastseg)
    cur0 = plsc.load_gather(smacc, [r0 + 4, b], mask=lastseg)
    plsc.store_scatter(smacc, [r0 + 4, b], jnp.maximum(cur0, m0), mask=lastseg)
    cur1 = plsc.load_gather(smacc, [r0 + 5, b], mask=lastseg)
    plsc.store_scatter(smacc, [r0 + 5, b], jnp.maximum(cur1, m1), mask=lastseg)
    return c

  lax.fori_loop(0, _ITER, _small_loop, 0)
  pltpu.sync_copy(smacc, out_sm.at[wid])


_sc_pool = functools.partial(
    pl.kernel,
    out_type=[
        jax.ShapeDtypeStruct((_NW, _G // 2, _NT), jnp.int32),
        jax.ShapeDtypeStruct((_NW, _G // 2, _NT), jnp.int32),
        jax.ShapeDtypeStruct((_NW, 16, _G), jnp.float32),
    ],
    mesh=plsc.VectorSubcoreMesh(core_axis_name="c", subcore_axis_name="s"),
    compiler_params=pltpu.CompilerParams(needs_layout_passes=False),
    scratch_types=[
        pltpu.VMEM((_SBUF,), jnp.int32),
        pltpu.VMEM((_SBUF,), jnp.int32),
        pltpu.VMEM((_SBUF,), jnp.int32),
        pltpu.VMEM((_SBUF,), jnp.float32),
        pltpu.VMEM((_SBUF,), jnp.float32),
        pltpu.VMEM((_G // 2, _NT), jnp.int32),
        pltpu.VMEM((_G // 2, _NT), jnp.int32),
        pltpu.VMEM((16, _G), jnp.float32),
    ],
)(_sc_pool_body)


def _leaky(v):
  return jnp.where(v > 0, v, 0.01 * v)


def _tc_mlp_body(ht, hk, sm, w1b, w1c, w1s, b1, w2, b2, w3, b3, out):
  wt = ht[...]
  wk = hk[...]
  lo_t = jnp.sum(jnp.bitwise_and(wt, 0xFFFF).astype(jnp.float32), axis=0)
  hi_t = jnp.sum(lax.shift_right_logical(wt, 16).astype(jnp.float32), axis=0)
  lo_k = jnp.sum(jnp.bitwise_and(wk, 0xFFFF).astype(jnp.float32), axis=0)
  hi_k = jnp.sum(lax.shift_right_logical(wk, 16).astype(jnp.float32), axis=0)
  smv = sm[...]
  sums = jnp.sum(smv[:, 0:4, :], axis=0)
  maxs = jnp.max(smv[:, 4:6, :], axis=0)
  cnt_r = jnp.maximum(jnp.sum(smv[:, 6:7, :], axis=0), 1.0)

  at = jnp.stack([lo_t, hi_t], axis=1).reshape(_G, _NT)
  ak = jnp.stack([lo_k, hi_k], axis=1).reshape(_G, _NT)
  cnt = jnp.sum(at, axis=1, keepdims=True)
  cntc = jnp.maximum(cnt, 1.0)
  empty = cnt <= 0.0
  mt = at / cntc
  mk = ak / cntc
  xt = jnp.where(empty, _NEG, (at > 0).astype(jnp.float32))
  xk = jnp.where(empty, _NEG, (ak > 0).astype(jnp.float32))
  st = jnp.sqrt(jnp.clip(mt - mt * mt, 0.0, None) + 1e-8)
  sk = jnp.sqrt(jnp.clip(mk - mk * mk, 0.0, None) + 1e-8)
  k128 = cnt - jnp.sum(ak, axis=1, keepdims=True)
  mk1 = k128 / cntc
  xk1 = jnp.where(empty, _NEG, (k128 > 0).astype(jnp.float32))
  sk1 = jnp.sqrt(jnp.clip(mk1 - mk1 * mk1, 0.0, None) + 1e-8)

  ms = sums[0:2, :] / cnt_r
  qs = sums[2:4, :] / cnt_r
  ss = jnp.sqrt(jnp.clip(qs - ms * ms, 0.0, None) + 1e-8)
  small_f = jnp.concatenate([ms, maxs, ss], axis=0)

  hbig = jnp.concatenate([mt, mk, xt, xk, st, sk], axis=1)
  hb = jnp.concatenate([mk1, xk1, sk1], axis=1)
  z1 = jnp.dot(hbig, w1b[...], preferred_element_type=jnp.float32,
               precision=lax.Precision.HIGHEST)
  z1 += jnp.dot(hb, w1c[...], preferred_element_type=jnp.float32,
                precision=lax.Precision.HIGHEST)
  z1 += lax.dot_general(small_f, w1s[...], (((0,), (0,)), ((), ())),
                        preferred_element_type=jnp.float32,
                        precision=lax.Precision.HIGHEST)
  h1 = _leaky(z1 + b1[...])
  h2 = _leaky(jnp.dot(h1, w2[...], preferred_element_type=jnp.float32,
                      precision=lax.Precision.HIGHEST) + b2[...])
  out[...] = jnp.sum(h2 * w3[...], axis=1, keepdims=True) + b3[...]


def _tc_mlp(ht, hk, sm, w1b, w1c, w1s, b1, w2, b2, w3row, b3):
  return pl.pallas_call(
      _tc_mlp_body,
      out_shape=jax.ShapeDtypeStruct((_G, 1), jnp.float32),
  )(ht, hk, sm, w1b, w1c, w1s, b1, w2, b2, w3row, b3)


def kernel(x_type, x_tok, x_small, batch, W1, b1, W2, b2, W3, b3):
  bat = batch.astype(jnp.int32)
  typ = x_type.astype(jnp.int32)
  tok = x_tok.astype(jnp.int32)
  xs = x_small.astype(jnp.float32)
  ht, hk, sm = _sc_pool(bat, typ, tok, xs[:, 0], xs[:, 1])
  out = _tc_mlp(
      ht,
      hk,
      sm,
      W1[_PERM_BIG],
      W1[_B128],
      W1[_SMALL],
      b1.reshape(1, 256),
      W2,
      b2.reshape(1, 256),
      W3.reshape(1, 256),
      b3.reshape(1, 1),
  )
  return out.reshape(-1)


# speedup vs baseline: 2.4028x; 1.0450x over previous
import functools

import numpy as np

import jax
import jax.numpy as jnp
from jax import lax
from jax.experimental import pallas as pl
from jax.experimental.pallas import tpu as pltpu
from jax.experimental.pallas import tpu_sc as plsc

_N = 160000
_G = 512
_NT = 128
_NK = 129
_NW = 32
_CHUNK = _N // _NW
_SBUF = 5008
_ITER = _SBUF // 16
_NEG = float("-inf")


_PERM_BIG = np.concatenate([np.arange(b, b + _NT) for b in
                            (0, 128, 259, 387, 518, 646)]).astype(np.int32)
_B128 = np.array([256, 515, 774], dtype=np.int32)
_SMALL = np.array([257, 258, 516, 517, 775, 776], dtype=np.int32)


def _take(x, idx):
  return x.at[idx].get(mode="promise_in_bounds")


def _sc_pool_body(bat_h, typ_h, tok_h, xs0_h, xs1_h,
                  out_t, out_k, out_sm,
                  b_v, t_v, k_v, s0_v, s1_v, hist_t, hist_k, smacc):
  cid = lax.axis_index("c")
  sid = lax.axis_index("s")
  wid = sid * 2 + cid
  base = wid * _CHUNK
  pltpu.sync_copy(bat_h.at[pl.ds(base, _CHUNK)], b_v.at[pl.ds(0, _CHUNK)])
  pltpu.sync_copy(typ_h.at[pl.ds(base, _CHUNK)], t_v.at[pl.ds(0, _CHUNK)])
  pltpu.sync_copy(tok_h.at[pl.ds(base, _CHUNK)], k_v.at[pl.ds(0, _CHUNK)])
  pltpu.sync_copy(xs0_h.at[pl.ds(base, _CHUNK)], s0_v.at[pl.ds(0, _CHUNK)])
  pltpu.sync_copy(xs1_h.at[pl.ds(base, _CHUNK)], s1_v.at[pl.ds(0, _CHUNK)])

  lanes = lax.iota(jnp.int32, 16)
  izero16 = jnp.zeros((16,), jnp.int32)
  zero16 = jnp.zeros((16,), jnp.float32)
  ninf16 = jnp.full((16,), _NEG, jnp.float32)

  def _zero_hists(j, c):
    for q in range(8):
      hist_t[j, pl.ds(q * 16, 16)] = izero16
      hist_k[j, pl.ds(q * 16, 16)] = izero16
    return c

  lax.fori_loop(0, _G // 2, _zero_hists, 0)
  for r in range(16):
    val = ninf16 if r in (4, 5) else zero16
    for q in range(_G // 16):
      smacc[r, pl.ds(q * 16, 16)] = val

  def _hist_loop(i, c):
    off = i * 16
    valid = (off + lanes) < _CHUNK
    b = jnp.where(valid, b_v[pl.ds(off, 16)], -1)
    t = t_v[pl.ds(off, 16)]
    k = jnp.clip(k_v[pl.ds(off, 16)], 0, _NK - 1)
    odd = (b & 1) == 1
    even = valid & jnp.logical_not(odd)
    oddm = valid & odd
    row = lax.shift_right_logical(b, 1)
    one = lanes * 0 + 1
    big = lax.shift_left(one, 16)
    plsc.addupdate_scatter(hist_t, [row, t], one, mask=even)
    plsc.addupdate_scatter(hist_t, [row, t], big, mask=oddm)
    kin = k < _NT
    plsc.addupdate_scatter(hist_k, [row, k], one, mask=even & kin)
    plsc.addupdate_scatter(hist_k, [row, k], big, mask=oddm & kin)
    return c

  lax.fori_loop(0, _ITER, _hist_loop, 0)
  pltpu.sync_copy(hist_t, out_t.at[wid])
  pltpu.sync_copy(hist_k, out_k.at[wid])

  def _small_loop(i, c):
    off = i * 16
    valid = (off + lanes) < _CHUNK
    b = jnp.where(valid, b_v[pl.ds(off, 16)], -1)
    v0 = jnp.where(valid, s0_v[pl.ds(off, 16)], 0.0)
    v1 = jnp.where(valid, s1_v[pl.ds(off, 16)], 0.0)
    s0 = v0
    s1 = v1
    q0 = v0 * v0
    q1 = v1 * v1
    m0 = jnp.where(valid, v0, _NEG)
    m1 = jnp.where(valid, v1, _NEG)
    for d in (1, 2, 4, 8):
      sidx = jnp.maximum(lanes - d, 0)
      bd = _take(b, sidx)
      same = (bd == b) & (lanes >= d)
      s0 = s0 + jnp.where(same, _take(s0, sidx), 0.0)
      s1 = s1 + jnp.where(same, _take(s1, sidx), 0.0)
      q0 = q0 + jnp.where(same, _take(q0, sidx), 0.0)
      q1 = q1 + jnp.where(same, _take(q1, sidx), 0.0)
      m0 = jnp.maximum(m0, jnp.where(same, _take(m0, sidx), _NEG))
      m1 = jnp.maximum(m1, jnp.where(same, _take(m1, sidx), _NEG))
    nb = _take(b, jnp.minimum(lanes + 1, 15))
    lastseg = ((b != nb) | (lanes == 15)) & valid
    r0 = lanes * 0
    plsc.addupdate_scatter(smacc, [r0, b], s0, mask=lastseg)
    plsc.addupdate_scatter(smacc, [r0 + 1, b], s1, mask=lastseg)
    plsc.addupdate_scatter(smacc, [r0 + 2, b], q0, mask=lastseg)
    plsc.addupdate_scatter(smacc, [r0 + 3, b], q1, mask=lastseg)
    cur0 = plsc.load_gather(smacc, [r0 + 4, b], mask=lastseg)
    plsc.store_scatter(smacc, [r0 + 4, b], jnp.maximum(cur0, m0), mask=lastseg)
    cur1 = plsc.load_gather(smacc, [r0 + 5, b], mask=lastseg)
    plsc.store_scatter(smacc, [r0 + 5, b], jnp.maximum(cur1, m1), mask=lastseg)
    return c

  lax.fori_loop(0, _ITER, _small_loop, 0)
  pltpu.sync_copy(smacc, out_sm.at[wid])


_sc_pool = functools.partial(
    pl.kernel,
    out_type=[
        jax.ShapeDtypeStruct((_NW, _G // 2, _NT), jnp.int32),
        jax.ShapeDtypeStruct((_NW, _G // 2, _NT), jnp.int32),
        jax.ShapeDtypeStruct((_NW, 16, _G), jnp.float32),
    ],
    mesh=plsc.VectorSubcoreMesh(core_axis_name="c", subcore_axis_name="s"),
    compiler_params=pltpu.CompilerParams(needs_layout_passes=False),
    scratch_types=[
        pltpu.VMEM((_SBUF,), jnp.int32),
        pltpu.VMEM((_SBUF,), jnp.int32),
        pltpu.VMEM((_SBUF,), jnp.int32),
        pltpu.VMEM((_SBUF,), jnp.float32),
        pltpu.VMEM((_SBUF,), jnp.float32),
        pltpu.VMEM((_G // 2, _NT), jnp.int32),
        pltpu.VMEM((_G // 2, _NT), jnp.int32),
        pltpu.VMEM((16, _G), jnp.float32),
    ],
)(_sc_pool_body)


def _leaky(v):
  return jnp.where(v > 0, v, 0.01 * v)


def _tc_mlp_body(ht, hk, sm, w1b, w1c, w1s, b1, w2, b2, w3, b3, out):
  wt = ht[...]
  wk = hk[...]
  lo_t = jnp.sum(jnp.bitwise_and(wt, 0xFFFF).astype(jnp.float32), axis=0)
  hi_t = jnp.sum(lax.shift_right_logical(wt, 16).astype(jnp.float32), axis=0)
  lo_k = jnp.sum(jnp.bitwise_and(wk, 0xFFFF).astype(jnp.float32), axis=0)
  hi_k = jnp.sum(lax.shift_right_logical(wk, 16).astype(jnp.float32), axis=0)
  smv = sm[...]
  sums = jnp.sum(smv[:, 0:4, :], axis=0)
  maxs = jnp.max(smv[:, 4:6, :], axis=0)

  at = jnp.stack([lo_t, hi_t], axis=1).reshape(_G, _NT)
  ak = jnp.stack([lo_k, hi_k], axis=1).reshape(_G, _NT)
  cnt = jnp.sum(at, axis=1, keepdims=True)
  cntc = jnp.maximum(cnt, 1.0)
  ones_row = jnp.ones((1, _NT), jnp.float32)
  cnt_r = jnp.maximum(
      lax.dot_general(ones_row, at, (((1,), (1,)), ((), ())),
                      preferred_element_type=jnp.float32,
                      precision=lax.Precision.HIGHEST), 1.0)
  empty = cnt <= 0.0
  mt = at / cntc
  mk = ak / cntc
  xt = jnp.where(empty, _NEG, (at > 0).astype(jnp.float32))
  xk = jnp.where(empty, _NEG, (ak > 0).astype(jnp.float32))
  st = jnp.sqrt(jnp.clip(mt - mt * mt, 0.0, None) + 1e-8)
  sk = jnp.sqrt(jnp.clip(mk - mk * mk, 0.0, None) + 1e-8)
  k128 = cnt - jnp.sum(ak, axis=1, keepdims=True)
  mk1 = k128 / cntc
  xk1 = jnp.where(empty, _NEG, (k128 > 0).astype(jnp.float32))
  sk1 = jnp.sqrt(jnp.clip(mk1 - mk1 * mk1, 0.0, None) + 1e-8)

  ms = sums[0:2, :] / cnt_r
  qs = sums[2:4, :] / cnt_r
  ss = jnp.sqrt(jnp.clip(qs - ms * ms, 0.0, None) + 1e-8)
  small_f = jnp.concatenate([ms, maxs, ss], axis=0)

  hbig = jnp.concatenate([mt, mk, xt, xk, st, sk], axis=1)
  hb = jnp.concatenate([mk1, xk1, sk1], axis=1)
  z1 = jnp.dot(hbig, w1b[...], preferred_element_type=jnp.float32,
               precision=lax.Precision.HIGHEST)
  z1 += jnp.dot(hb, w1c[...], preferred_element_type=jnp.float32,
                precision=lax.Precision.HIGHEST)
  z1 += lax.dot_general(small_f, w1s[...], (((0,), (0,)), ((), ())),
                        preferred_element_type=jnp.float32,
                        precision=lax.Precision.HIGHEST)
  h1 = _leaky(z1 + b1[...])
  h2 = _leaky(jnp.dot(h1, w2[...], preferred_element_type=jnp.float32,
                      precision=lax.Precision.HIGHEST) + b2[...])
  out[...] = jnp.sum(h2 * w3[...], axis=1, keepdims=True) + b3[...]


def _tc_mlp(ht, hk, sm, w1b, w1c, w1s, b1, w2, b2, w3row, b3):
  return pl.pallas_call(
      _tc_mlp_body,
      out_shape=jax.ShapeDtypeStruct((_G, 1), jnp.float32),
  )(ht, hk, sm, w1b, w1c, w1s, b1, w2, b2, w3row, b3)


def kernel(x_type, x_tok, x_small, batch, W1, b1, W2, b2, W3, b3):
  bat = batch.astype(jnp.int32)
  typ = x_type.astype(jnp.int32)
  tok = x_tok.astype(jnp.int32)
  xs = x_small.astype(jnp.float32)
  ht, hk, sm = _sc_pool(bat, typ, tok, xs[:, 0], xs[:, 1])
  out = _tc_mlp(
      ht,
      hk,
      sm,
      W1[_PERM_BIG],
      W1[_B128],
      W1[_SMALL],
      b1.reshape(1, 256),
      W2,
      b2.reshape(1, 256),
      W3.reshape(1, 256),
      b3.reshape(1, 1),
  )
  return out.reshape(-1)
